# baseline (speedup 1.0000x reference)
import functools

import jax
import jax.numpy as jnp
from jax import lax
from jax.experimental import pallas as pl
from jax.experimental.pallas import tpu as pltpu
from jax.experimental.pallas import tpu_sc as plsc

N = 10000
E = 320000
D = 128
HD = D // 2
NUM_CONVS = 3

NC = 2
NS = 16
NW = NC * NS

NP = 10240
NPX = 10016
CHUNK = 128
NCHUNKS = 84
EPT = NCHUNKS * CHUNK
E_PAD = NW * EPT
ROWS_PER_TILE = NPX // NS
EMB_ROWS_PER_TILE = NP // NW
EMB_CHUNK = 80


@functools.cache
def _mesh():
    return plsc.VectorSubcoreMesh(
        core_axis_name="c", subcore_axis_name="s",
        num_cores=NC, num_subcores=NS)


@functools.cache
def _embed_kernel():
    return pl.kernel(
        _embed_body,
        out_type=jax.ShapeDtypeStruct((NP, D), jnp.float32),
        mesh=_mesh(),
        scratch_types=[
            pltpu.VMEM((EMB_CHUNK,), jnp.int32),
            pltpu.VMEM((EMB_CHUNK, D), jnp.float32),
            pltpu.SemaphoreType.DMA,
        ],
    )


def _embed_body(emb_hbm, z_hbm, out_hbm, idx_v, rows_v, sem):
    cid = lax.axis_index("c")
    sid = lax.axis_index("s")
    wid = sid * NC + cid
    base = pl.multiple_of(wid * EMB_ROWS_PER_TILE, 8)
    for i in range(EMB_ROWS_PER_TILE // EMB_CHUNK):
        b = pl.multiple_of(base + i * EMB_CHUNK, 8)
        pltpu.sync_copy(z_hbm.at[pl.ds(b, EMB_CHUNK)], idx_v)
        pltpu.async_copy(emb_hbm.at[idx_v], rows_v, sem).wait()
        pltpu.sync_copy(rows_v, out_hbm.at[pl.ds(b, EMB_CHUNK)])


@functools.cache
def _conv_kernel():
    return pl.kernel(
        _conv_body,
        out_type=[jax.ShapeDtypeStruct((NPX, HD), jnp.float32),
                  jax.ShapeDtypeStruct((NPX, HD), jnp.float32),
                  jax.ShapeDtypeStruct((NPX, HD), jnp.float32),
                  jax.ShapeDtypeStruct((NPX, HD), jnp.float32)],
        mesh=_mesh(),
        compiler_params=pltpu.CompilerParams(use_tc_tiling_on_sc=False),
        scratch_types=[
            [pltpu.VMEM((2, CHUNK), jnp.int32) for _ in range(6)],
            pltpu.VMEM((CHUNK, HD), jnp.float32),
            pltpu.VMEM((CHUNK, HD), jnp.float32),
            pltpu.VMEM((CHUNK, HD), jnp.float32),
            pltpu.VMEM((CHUNK, HD), jnp.float32),
            pltpu.VMEM((CHUNK, HD), jnp.float32),
            pltpu.VMEM((CHUNK, HD), jnp.float32),
            pltpu.VMEM_SHARED((NPX, HD), jnp.float32),
            pltpu.VMEM_SHARED((NPX, HD), jnp.float32),
            [pltpu.SemaphoreType.DMA for _ in range(2)],
            [pltpu.SemaphoreType.DMA for _ in range(2)],
            [pltpu.SemaphoreType.DMA for _ in range(2)],
            [pltpu.SemaphoreType.DMA for _ in range(6)],
        ],
    )


def _conv_body(xa_hbm, xb_hbm, idx_hbm, eaa_hbm, eab_hbm,
               zeros_hbm,
               out0a_hbm, out0b_hbm, out1a_hbm, out1b_hbm,
               idxb, rin0, rin1, ea0, ea1, msg0, msg1,
               xtab, acc, sg, se, ss, si):
    cid = lax.axis_index("c")
    sid = lax.axis_index("s")
    wid = sid * NC + cid
    rin = (rin0, rin1)
    ea = (ea0, ea1)
    msg = (msg0, msg1)

    r0 = sid * ROWS_PER_TILE
    ebase = wid * EPT

    def ea_base(i):
        return pl.multiple_of(jnp.minimum(ebase + i * CHUNK, E - CHUNK), 8)

    for xt_hbm, eat_hbm, o0_hbm, o1_hbm in (
            (xa_hbm, eaa_hbm, out0a_hbm, out1a_hbm),
            (xb_hbm, eab_hbm, out0b_hbm, out1b_hbm)):
        pltpu.sync_copy(xt_hbm.at[pl.ds(r0, ROWS_PER_TILE)],
                        xtab.at[pl.ds(r0, ROWS_PER_TILE)])
        pltpu.sync_copy(zeros_hbm, acc.at[pl.ds(r0, ROWS_PER_TILE)])
        plsc.subcore_barrier()

        def load_idx(i, q):
            pltpu.async_copy(idx_hbm.at[wid * NCHUNKS + i], idxb[q],
                             si[q])

        def wait_idx(q):
            pltpu.make_async_copy(idx_hbm.at[0], idxb[q], si[q]).wait()

        def issue_gather(q, b2):
            pltpu.async_copy(xtab.at[idxb[q].at[0]], rin[b2], sg[b2])

        def issue_ea(i, b2):
            pltpu.async_copy(eat_hbm.at[pl.ds(ea_base(i), CHUNK)], ea[b2],
                             se[b2])

        for q in range(4):
            load_idx(q, q)
        for b in range(2):
            wait_idx(b)
            issue_gather(b, b)
            issue_ea(b, b)

        def step(k, t):
            i = 6 * k + t
            b2 = t % 2

            def wait_ss():
                pltpu.make_async_copy(msg[b2], acc.at[idxb[0].at[1]],
                                      ss[b2]).wait()
            if t < 2:
                pl.when(k >= 1)(wait_ss)
            else:
                wait_ss()

            def _idx():
                load_idx(i + 4, (t + 4) % 6)
            if t < 2:
                _idx()
            else:
                pl.when(k < NCHUNKS // 6 - 1)(_idx)

            pltpu.make_async_copy(xtab.at[idxb[0].at[0]], rin[b2],
                                  sg[b2]).wait()
            pltpu.make_async_copy(eat_hbm.at[pl.ds(0, CHUNK)], ea[b2],
                                  se[b2]).wait()

            def crow(r, c):
                for jj in range(HD // 16):
                    sl = pl.ds(jj * 16, 16)
                    msg[b2][r, sl] = jnp.maximum(
                        rin[b2][r, sl] + ea[b2][r, sl], 0.0)
                return c
            lax.fori_loop(0, CHUNK, crow, 0, unroll=4)

            pltpu.async_copy(msg[b2], acc.at[idxb[t % 6].at[1]], ss[b2],
                             add=True)

            def _gea():
                wait_idx((t + 2) % 6)
                issue_gather((t + 2) % 6, b2)
                issue_ea(i + 2, b2)
            if t < 4:
                _gea()
            else:
                pl.when(k < NCHUNKS // 6 - 1)(_gea)

        def sext(k, carry):
            for t in range(6):
                step(k, t)
            return carry
        lax.fori_loop(0, NCHUNKS // 6, sext, 0)

        for b2 in range(2):
            pltpu.make_async_copy(msg[b2], acc.at[idxb[0].at[1]],
                                  ss[b2]).wait()

        plsc.subcore_barrier()

        @pl.when(cid == 0)
        def _():
            pltpu.sync_copy(acc.at[pl.ds(r0, ROWS_PER_TILE)],
                            o0_hbm.at[pl.ds(r0, ROWS_PER_TILE)])

        @pl.when(cid == 1)
        def _():
            pltpu.sync_copy(acc.at[pl.ds(r0, ROWS_PER_TILE)],
                            o1_hbm.at[pl.ds(r0, ROWS_PER_TILE)])
        plsc.subcore_barrier()


def _mlp_body(relu_mid, p0a_ref, p0b_ref, p1a_ref, p1b_ref, x_ref, w1_ref,
              b1_ref, w2_ref, b2_ref, o_ref):
    x = x_ref[...]
    agg = jnp.concatenate([p0a_ref[...] + p1a_ref[...],
                           p0b_ref[...] + p1b_ref[...]], axis=1)
    out = agg + x
    h = jnp.maximum(
        jnp.dot(out, w1_ref[...], preferred_element_type=jnp.float32)
        + b1_ref[...], 0.0)
    y = jnp.dot(h, w2_ref[...], preferred_element_type=jnp.float32) + b2_ref[...]
    if relu_mid:
        y = jnp.maximum(y, 0.0)
    o_ref[...] = y + x


def _mlp(parts, x, W1i, b1i, W2i, b2i, relu_mid):
    R = 2504
    row_spec = pl.BlockSpec((R, D), lambda i: (i, 0))
    half_spec = pl.BlockSpec((R, HD), lambda i: (i, 0))
    full2 = pl.BlockSpec((D, D), lambda i: (0, 0))
    bias = pl.BlockSpec((1, D), lambda i: (0, 0))
    return pl.pallas_call(
        functools.partial(_mlp_body, relu_mid),
        grid=(NPX // R,),
        in_specs=[half_spec, half_spec, half_spec, half_spec, row_spec,
                  full2, bias, full2, bias],
        out_specs=row_spec,
        out_shape=jax.ShapeDtypeStruct((NPX, D), jnp.float32),
    )(parts[0], parts[1], parts[2], parts[3], x,
      W1i, b1i.reshape(1, D), W2i, b2i.reshape(1, D))


def kernel(z, edge_index, edge_attr, emb, W1, b1, W2, b2):
    z_pad = jnp.concatenate(
        [z.astype(jnp.int32), jnp.zeros((NP - N,), jnp.int32)])
    src = edge_index[0].astype(jnp.int32)
    dst = edge_index[1].astype(jnp.int32)
    pad_e = E_PAD - E
    src_pad = jnp.concatenate([src, jnp.zeros((pad_e,), jnp.int32)])
    dst_pad = jnp.concatenate([dst, jnp.full((pad_e,), N, jnp.int32)])
    idx_comb = jnp.stack([src_pad.reshape(NW * NCHUNKS, CHUNK),
                          dst_pad.reshape(NW * NCHUNKS, CHUNK)],
                         axis=1)
    zeros_blk = jnp.zeros((ROWS_PER_TILE, HD), jnp.float32)

    ea_a = edge_attr[:, :HD]
    ea_b = edge_attr[:, HD:]
    x = _embed_kernel()(emb, z_pad)[:NPX]
    for i in range(NUM_CONVS):
        xa = x[:, :HD]
        xb = x[:, HD:]
        parts = _conv_kernel()(xa, xb, idx_comb, ea_a, ea_b, zeros_blk)
        x = _mlp(parts, x, W1[i], b1[i], W2[i], b2[i],
                 relu_mid=(i < NUM_CONVS - 1))
    return x[:N]


# speedup vs baseline: 1.0186x; 1.0186x over previous
import functools

import jax
import jax.numpy as jnp
from jax import lax
from jax.experimental import pallas as pl
from jax.experimental.pallas import tpu as pltpu
from jax.experimental.pallas import tpu_sc as plsc

N = 10000
E = 320000
D = 128
HD = D // 2
NUM_CONVS = 3

NC = 2
NS = 16
NW = NC * NS

NP = 10240
NPX = 10016
CHUNK = 128
NCHUNKS = 80
EPT = NCHUNKS * CHUNK
E_PAD = NW * EPT
ROWS_PER_TILE = NPX // NS
EMB_ROWS_PER_TILE = NP // NW
EMB_CHUNK = 80


@functools.cache
def _mesh():
    return plsc.VectorSubcoreMesh(
        core_axis_name="c", subcore_axis_name="s",
        num_cores=NC, num_subcores=NS)


@functools.cache
def _embed_kernel():
    return pl.kernel(
        _embed_body,
        out_type=jax.ShapeDtypeStruct((NP, D), jnp.float32),
        mesh=_mesh(),
        scratch_types=[
            pltpu.VMEM((EMB_CHUNK,), jnp.int32),
            pltpu.VMEM((EMB_CHUNK, D), jnp.float32),
            pltpu.SemaphoreType.DMA,
        ],
    )


def _embed_body(emb_hbm, z_hbm, out_hbm, idx_v, rows_v, sem):
    cid = lax.axis_index("c")
    sid = lax.axis_index("s")
    wid = sid * NC + cid
    base = pl.multiple_of(wid * EMB_ROWS_PER_TILE, 8)
    for i in range(EMB_ROWS_PER_TILE // EMB_CHUNK):
        b = pl.multiple_of(base + i * EMB_CHUNK, 8)
        pltpu.sync_copy(z_hbm.at[pl.ds(b, EMB_CHUNK)], idx_v)
        pltpu.async_copy(emb_hbm.at[idx_v], rows_v, sem).wait()
        pltpu.sync_copy(rows_v, out_hbm.at[pl.ds(b, EMB_CHUNK)])


@functools.cache
def _conv_kernel():
    return pl.kernel(
        _conv_body,
        out_type=[jax.ShapeDtypeStruct((NPX, HD), jnp.float32),
                  jax.ShapeDtypeStruct((NPX, HD), jnp.float32),
                  jax.ShapeDtypeStruct((NPX, HD), jnp.float32),
                  jax.ShapeDtypeStruct((NPX, HD), jnp.float32)],
        mesh=_mesh(),
        compiler_params=pltpu.CompilerParams(use_tc_tiling_on_sc=False),
        scratch_types=[
            [pltpu.VMEM((CHUNK,), jnp.int32) for _ in range(4)],
            [pltpu.VMEM((CHUNK,), jnp.int32) for _ in range(8)],
            pltpu.VMEM((CHUNK, HD), jnp.float32),
            pltpu.VMEM((CHUNK, HD), jnp.float32),
            pltpu.VMEM((CHUNK, HD), jnp.float32),
            pltpu.VMEM((CHUNK, HD), jnp.float32),
            pltpu.VMEM((CHUNK, HD), jnp.float32),
            pltpu.VMEM((CHUNK, HD), jnp.float32),
            pltpu.VMEM_SHARED((NPX, HD), jnp.float32),
            pltpu.VMEM_SHARED((NPX, HD), jnp.float32),
            [pltpu.SemaphoreType.DMA for _ in range(2)],
            [pltpu.SemaphoreType.DMA for _ in range(2)],
            [pltpu.SemaphoreType.DMA for _ in range(2)],
            [pltpu.SemaphoreType.DMA for _ in range(4)],
            [pltpu.SemaphoreType.DMA for _ in range(8)],
        ],
    )


def _conv_body(xa_hbm, xb_hbm, src_hbm, dst_hbm, eaa_hbm, eab_hbm,
               zeros_hbm,
               out0a_hbm, out0b_hbm, out1a_hbm, out1b_hbm,
               srcb, dstb, rin0, rin1, ea0, ea1, msg0, msg1,
               xtab, acc, sg, se, ss, si_s, si_d):
    cid = lax.axis_index("c")
    sid = lax.axis_index("s")
    wid = sid * NC + cid
    rin = (rin0, rin1)
    ea = (ea0, ea1)
    msg = (msg0, msg1)

    r0 = sid * ROWS_PER_TILE
    ebase = wid * EPT

    def ea_base(i):
        return pl.multiple_of(jnp.minimum(ebase + i * CHUNK, E - CHUNK), 8)

    for xt_hbm, eat_hbm, o0_hbm, o1_hbm in (
            (xa_hbm, eaa_hbm, out0a_hbm, out1a_hbm),
            (xb_hbm, eab_hbm, out0b_hbm, out1b_hbm)):
        pltpu.sync_copy(xt_hbm.at[pl.ds(r0, ROWS_PER_TILE)],
                        xtab.at[pl.ds(r0, ROWS_PER_TILE)])
        pltpu.sync_copy(zeros_hbm, acc.at[pl.ds(r0, ROWS_PER_TILE)])
        plsc.subcore_barrier()

        def load_src(i, q):
            pltpu.async_copy(
                src_hbm.at[pl.ds(pl.multiple_of(ebase + i * CHUNK, 8),
                                 CHUNK)],
                srcb[q], si_s[q])

        def load_dst(i, q):
            pltpu.async_copy(
                dst_hbm.at[pl.ds(pl.multiple_of(ebase + i * CHUNK, 8),
                                 CHUNK)],
                dstb[q], si_d[q])

        def issue_gather(q, b2):
            pltpu.async_copy(xtab.at[srcb[q]], rin[b2], sg[b2])

        def issue_ea(i, b2):
            pltpu.async_copy(eat_hbm.at[pl.ds(ea_base(i), CHUNK)], ea[b2],
                             se[b2])

        for q in range(4):
            load_src(q, q)
            load_dst(q, q)
        for b in range(2):
            pltpu.make_async_copy(src_hbm.at[pl.ds(0, CHUNK)], srcb[b],
                                  si_s[b]).wait()
            issue_gather(b, b)
            issue_ea(b, b)

        def step(k, t):
            i = 8 * k + t
            b2 = t % 2

            def wait_ss():
                pltpu.make_async_copy(msg[b2], acc.at[dstb[0]],
                                      ss[b2]).wait()
            if t < 2:
                pl.when(k >= 1)(wait_ss)
            else:
                wait_ss()

            def _dst():
                load_dst(i + 4, (t + 4) % 8)
            if t < 4:
                _dst()
            else:
                pl.when(k < NCHUNKS // 8 - 1)(_dst)

            pltpu.make_async_copy(xtab.at[srcb[0]], rin[b2], sg[b2]).wait()
            pltpu.make_async_copy(eat_hbm.at[pl.ds(0, CHUNK)], ea[b2],
                                  se[b2]).wait()

            def crow(r, c):
                for jj in range(HD // 16):
                    sl = pl.ds(jj * 16, 16)
                    msg[b2][r, sl] = jnp.maximum(
                        rin[b2][r, sl] + ea[b2][r, sl], 0.0)
                return c
            lax.fori_loop(0, CHUNK, crow, 0, unroll=4)

            pltpu.make_async_copy(dst_hbm.at[pl.ds(0, CHUNK)], dstb[t],
                                  si_d[t]).wait()
            pltpu.async_copy(msg[b2], acc.at[dstb[t]], ss[b2], add=True)

            def _gea():
                pltpu.make_async_copy(src_hbm.at[pl.ds(0, CHUNK)],
                                      srcb[(t + 2) % 4],
                                      si_s[(t + 2) % 4]).wait()
                issue_gather((t + 2) % 4, b2)
                issue_ea(i + 2, b2)
            if t < 6:
                _gea()
            else:
                pl.when(k < NCHUNKS // 8 - 1)(_gea)

            def _src():
                load_src(i + 4, t % 4)
            if t < 4:
                _src()
            else:
                pl.when(k < NCHUNKS // 8 - 1)(_src)

        def octet(k, carry):
            for t in range(8):
                step(k, t)
            return carry
        lax.fori_loop(0, NCHUNKS // 8, octet, 0)

        for b2 in range(2):
            pltpu.make_async_copy(msg[b2], acc.at[dstb[0]], ss[b2]).wait()

        plsc.subcore_barrier()

        @pl.when(cid == 0)
        def _():
            pltpu.sync_copy(acc.at[pl.ds(r0, ROWS_PER_TILE)],
                            o0_hbm.at[pl.ds(r0, ROWS_PER_TILE)])

        @pl.when(cid == 1)
        def _():
            pltpu.sync_copy(acc.at[pl.ds(r0, ROWS_PER_TILE)],
                            o1_hbm.at[pl.ds(r0, ROWS_PER_TILE)])
        plsc.subcore_barrier()


def _mlp_body(relu_mid, p0a_ref, p0b_ref, p1a_ref, p1b_ref, x_ref, w1_ref,
              b1_ref, w2_ref, b2_ref, o_ref):
    x = x_ref[...]
    agg = jnp.concatenate([p0a_ref[...] + p1a_ref[...],
                           p0b_ref[...] + p1b_ref[...]], axis=1)
    out = agg + x
    h = jnp.maximum(
        jnp.dot(out, w1_ref[...], preferred_element_type=jnp.float32)
        + b1_ref[...], 0.0)
    y = jnp.dot(h, w2_ref[...], preferred_element_type=jnp.float32) + b2_ref[...]
    if relu_mid:
        y = jnp.maximum(y, 0.0)
    o_ref[...] = y + x


def _mlp(parts, x, W1i, b1i, W2i, b2i, relu_mid):
    R = 2504
    row_spec = pl.BlockSpec((R, D), lambda i: (i, 0))
    half_spec = pl.BlockSpec((R, HD), lambda i: (i, 0))
    full2 = pl.BlockSpec((D, D), lambda i: (0, 0))
    bias = pl.BlockSpec((1, D), lambda i: (0, 0))
    return pl.pallas_call(
        functools.partial(_mlp_body, relu_mid),
        grid=(NPX // R,),
        in_specs=[half_spec, half_spec, half_spec, half_spec, row_spec,
                  full2, bias, full2, bias],
        out_specs=row_spec,
        out_shape=jax.ShapeDtypeStruct((NPX, D), jnp.float32),
    )(parts[0], parts[1], parts[2], parts[3], x,
      W1i, b1i.reshape(1, D), W2i, b2i.reshape(1, D))


def kernel(z, edge_index, edge_attr, emb, W1, b1, W2, b2):
    z_pad = jnp.concatenate(
        [z.astype(jnp.int32), jnp.zeros((NP - N,), jnp.int32)])
    src = edge_index[0].astype(jnp.int32)
    dst = edge_index[1].astype(jnp.int32)
    pad_e = E_PAD - E
    src_pad = jnp.concatenate([src, jnp.zeros((pad_e,), jnp.int32)])
    dst_pad = jnp.concatenate([dst, jnp.full((pad_e,), N, jnp.int32)])
    zeros_blk = jnp.zeros((ROWS_PER_TILE, HD), jnp.float32)

    ea_a = edge_attr[:, :HD]
    ea_b = edge_attr[:, HD:]
    x = _embed_kernel()(emb, z_pad)[:NPX]
    for i in range(NUM_CONVS):
        xa = x[:, :HD]
        xb = x[:, HD:]
        parts = _conv_kernel()(xa, xb, src_pad, dst_pad, ea_a, ea_b,
                               zeros_blk)
        x = _mlp(parts, x, W1[i], b1[i], W2[i], b2[i],
                 relu_mid=(i < NUM_CONVS - 1))
    return x[:N]
